# baseline (speedup 1.0000x reference)
import math

import jax
import jax.numpy as jnp
from jax.experimental import pallas as pl

_ANCHORS = [[(10.0, 13.0), (16.0, 30.0), (33.0, 23.0)],
            [(30.0, 61.0), (62.0, 45.0), (59.0, 119.0)],
            [(116.0, 90.0), (156.0, 198.0), (373.0, 326.0)]]
_STRIDES = [8.0, 16.0, 32.0]
_NC = 80
_NA = 3


def _bt(shapes, targets):
    anchors = jnp.array(_ANCHORS, dtype=jnp.float32)
    nt = targets.shape[0]
    ai = jnp.tile(jnp.arange(_NA, dtype=jnp.float32)[:, None], (1, nt))
    t_all = jnp.concatenate([jnp.tile(targets[None], (_NA, 1, 1)), ai[:, :, None]], axis=2)
    offset = 0.5 * jnp.array([[0, 0], [1, 0], [0, 1], [-1, 0], [0, -1]], dtype=jnp.float32)
    tcls, tbox, indices, anch = [], [], [], []
    for s, stride in enumerate(_STRIDES):
        W = float(shapes[s][3]); H = float(shapes[s][2])
        gain = jnp.array([1.0, 1.0, W, H, W, H, 1.0], dtype=jnp.float32)
        t = t_all * gain
        anchor_s = anchors[s] / stride
        r = t[:, :, 4:6] / anchor_s[:, None]
        jmask0 = jnp.maximum(r, 1.0 / r).max(axis=2) < 4.0
        t = t.reshape(_NA * nt, 7)
        mask0 = jmask0.reshape(_NA * nt)
        gxy = t[:, 2:4]
        gxi = gain[2:4] - gxy
        c1 = (gxy % 1.0 < 0.5) & (gxy > 1.0)
        c2 = (gxi % 1.0 < 0.5) & (gxi > 1.0)
        jj, kk = c1[:, 0], c1[:, 1]
        ll, mm = c2[:, 0], c2[:, 1]
        jm = jnp.stack([jnp.ones_like(jj), jj, kk, ll, mm])
        mask = (jm & mask0[None]).reshape(5 * _NA * nt)
        t = jnp.tile(t, (5, 1, 1)).reshape(5 * _NA * nt, 7)
        offsets = (jnp.zeros_like(gxy)[None] + offset[:, None]).reshape(5 * _NA * nt, 2)
        b = t[:, 0].astype(jnp.int32)
        c = t[:, 1].astype(jnp.int32)
        gxy = t[:, 2:4]
        gwh = t[:, 4:6]
        gij = (gxy - offsets).astype(jnp.int32)
        gi, gj = gij[:, 0], gij[:, 1]
        a = t[:, 6].astype(jnp.int32)
        indices.append((b, a, gj, gi, mask))
        tbox.append(jnp.concatenate([gxy - gij.astype(jnp.float32), gwh], axis=1))
        anch.append(anchor_s[a])
        tcls.append(c)
    return tcls, tbox, indices, anch


def _ciou(box1, box2):
    box2 = box2.T
    b1_x1 = box1[0] - box1[2] / 2.0; b1_x2 = box1[0] + box1[2] / 2.0
    b1_y1 = box1[1] - box1[3] / 2.0; b1_y2 = box1[1] + box1[3] / 2.0
    b2_x1 = box2[0] - box2[2] / 2.0; b2_x2 = box2[0] + box2[2] / 2.0
    b2_y1 = box2[1] - box2[3] / 2.0; b2_y2 = box2[1] + box2[3] / 2.0
    inter = jnp.maximum(jnp.minimum(b1_x2, b2_x2) - jnp.maximum(b1_x1, b2_x1), 0.0) * jnp.maximum(jnp.minimum(b1_y2, b2_y2) - jnp.maximum(b1_y1, b2_y1), 0.0)
    w1, h1 = b1_x2 - b1_x1, b1_y2 - b1_y1
    w2, h2 = b2_x2 - b2_x1, b2_y2 - b2_y1
    union = w1 * h1 + 1e-16 + w2 * h2 - inter
    iou = inter / union
    cw = jnp.maximum(b1_x2, b2_x2) - jnp.minimum(b1_x1, b2_x1)
    ch = jnp.maximum(b1_y2, b2_y2) - jnp.minimum(b1_y1, b2_y1)
    c2 = cw ** 2 + ch ** 2 + 1e-16
    rho2 = (b2_x1 + b2_x2 - (b1_x1 + b1_x2)) ** 2 / 4.0 + (b2_y1 + b2_y2 - (b1_y1 + b1_y2)) ** 2 / 4.0
    v = 4.0 / math.pi ** 2 * (jnp.arctan(w2 / h2) - jnp.arctan(w1 / h1)) ** 2
    alpha = jax.lax.stop_gradient(v / (1.0 - iou + v + 1e-16))
    return iou - (rho2 / c2 + v * alpha)


def _bce_full(p, t):
    logp = jnp.maximum(jnp.log(p), -100.0)
    log1p = jnp.maximum(jnp.log(1.0 - p), -100.0)
    return jnp.mean(-(t * logp + (1.0 - t) * log1p))


def _loss(preds, tcls, tbox, indices, anchs):
    lcls = jnp.zeros(1); lbox = jnp.zeros(1); lobj = jnp.zeros(1)
    balance = [4.0, 1.0, 0.4]
    for s, p in enumerate(preds):
        sh = p.shape
        no = sh[1] // _NA
        ps_all = p.reshape(sh[0], _NA, no, sh[2], sh[3]).transpose(0, 1, 3, 4, 2)
        b, a, gj, gi, mask = indices[s]
        n = int(b.shape[0])
        cnt = jnp.sum(mask.astype(jnp.float32))
        den = jnp.maximum(cnt, 1.0)
        ps = ps_all[b, a, gj, gi]
        pxy = ps[:, :2] * 2.0 - 0.5
        pwh = (ps[:, 2:4] * 2.0) ** 2 * anchs[s]
        pbox = jnp.concatenate([pxy, pwh], axis=1)
        iou = _ciou(pbox.T, tbox[s])
        d = jnp.abs(pbox - tbox[s])
        sl1 = jnp.where(d < 1.0, 0.5 * d * d, d - 0.5)
        sl1 = jnp.where(mask[:, None], sl1, 0.0)
        lbox = lbox + jnp.sum(sl1) / (den * 4.0)
        val = 0.5 + 0.5 * jnp.maximum(jax.lax.stop_gradient(iou), 0.0)
        tobj_ext = jnp.zeros((sh[0] + 1,) + ps_all[..., 0].shape[1:], dtype=jnp.float32)
        b_safe = jnp.where(mask, b, sh[0])
        tobj = tobj_ext.at[b_safe, a, gj, gi].set(val)[: sh[0]]
        tmat = jnp.zeros((n, no - 5), dtype=jnp.float32)
        tmat = tmat.at[jnp.arange(n), tcls[s]].set(1.0)
        pc = ps[:, 5:]
        logp = jnp.maximum(jnp.log(pc), -100.0)
        log1p = jnp.maximum(jnp.log(1.0 - pc), -100.0)
        bce = -(tmat * logp + (1.0 - tmat) * log1p)
        bce = jnp.where(mask[:, None], bce, 0.0)
        lcls = lcls + jnp.sum(bce) / (den * float(no - 5))
        lobj = lobj + _bce_full(ps_all[..., 4], tobj) * balance[s]
    lbox = lbox * 0.05
    lobj = lobj * 1.0 * 1.4
    lcls = lcls * 0.5
    bs = preds[0].shape[0]
    loss = (lbox + lobj + lcls) * bs
    return loss, jax.lax.stop_gradient(jnp.concatenate([lbox, lobj, lcls]))


def _identity_kernel(x_ref, o_ref):
    o_ref[...] = x_ref[...]


def kernel(pred0, pred1, pred2, targets):
    preds = [pred0, pred1, pred2]
    tcls, tbox, indices, anchs = _bt([p.shape for p in preds], targets)
    loss, parts = _loss(preds, tcls, tbox, indices, anchs)
    both = jnp.concatenate([loss, parts])
    both = pl.pallas_call(
        _identity_kernel,
        out_shape=jax.ShapeDtypeStruct((4,), jnp.float32),
    )(both)
    return both[:1], both[1:]


# speedup vs baseline: 3.2895x; 3.2895x over previous
import functools
import math

import jax
import jax.numpy as jnp
from jax.experimental import pallas as pl
from jax.experimental.pallas import tpu as pltpu

_INTERPRET = False

B = 16
NA = 3
NC = 80
NT = 4096
HWS = [4096, 1024, 256]
WS = [64, 32, 16]
NPOS = [196608, 49152, 12288]
MAPOFF = [0, 196608, 245760]
TOT = 258048
TOBJ_PAD = 258064
DUMMY = 258048
NE = 61440
BAL = [4.0, 1.0, 0.4]
ANCHORS_RAW = [[(10.0, 13.0), (16.0, 30.0), (33.0, 23.0)],
               [(30.0, 61.0), (62.0, 45.0), (59.0, 119.0)],
               [(116.0, 90.0), (156.0, 198.0), (373.0, 326.0)]]
STRIDES = [8.0, 16.0, 32.0]
ANCH = [[(a / s, b / s) for (a, b) in ANCHORS_RAW[i]] for i, s in enumerate(STRIDES)]
OFFS = [(0.0, 0.0), (0.5, 0.0), (0.0, 0.5), (-0.5, 0.0), (0.0, -0.5)]


def _prep_body(t_ref, idx_ref, f_ref):
    img = t_ref[0]
    cls = t_ref[1]
    x = t_ref[2]
    y = t_ref[3]
    w = t_ref[4]
    h = t_ref[5]
    b = img.astype(jnp.int32)
    tc = cls.astype(jnp.int32)
    for s in range(3):
        W = float(WS[s])
        HW = HWS[s]
        gx = x * W
        gy = y * W
        gw = w * W
        gh = h * W
        fx = gx - jnp.floor(gx)
        fy = gy - jnp.floor(gy)
        jj = (fx < 0.5) & (gx > 1.0)
        kk = (fy < 0.5) & (gy > 1.0)
        gxi = W - gx
        gyi = W - gy
        fxi = gxi - jnp.floor(gxi)
        fyi = gyi - jnp.floor(gyi)
        ll = (fxi < 0.5) & (gxi > 1.0)
        mm = (fyi < 0.5) & (gyi > 1.0)
        gates = [None, jj, kk, ll, mm]
        m0 = []
        for a in range(NA):
            aw, ah = ANCH[s][a]
            rw = gw * (1.0 / aw)
            rh = gh * (1.0 / ah)
            mw = jnp.maximum(rw, 1.0 / rw)
            mh = jnp.maximum(rh, 1.0 / rh)
            m0.append(jnp.maximum(mw, mh) < 4.0)
        for o in range(5):
            ox, oy = OFFS[o]
            gi = (gx - ox).astype(jnp.int32)
            gj = (gy - oy).astype(jnp.int32)
            gi = jnp.clip(gi, 0, WS[s] - 1)
            gj = jnp.clip(gj, 0, WS[s] - 1)
            tx = gx - gi.astype(jnp.float32)
            ty = gy - gj.astype(jnp.float32)
            pos = gj * WS[s] + gi
            for a in range(NA):
                ci = o * NA + a
                base = (b * 255 + 85 * a) * HW + pos
                for c in range(4):
                    idx_ref[s, c, ci] = base + c * HW
                idx_ref[s, 4, ci] = base + (5 + tc) * HW
                idx_ref[s, 5, ci] = (b * NA + a) * HW + pos + MAPOFF[s]
                if gates[o] is None:
                    mk = m0[a]
                else:
                    mk = gates[o] & m0[a]
                f_ref[s, 0, ci] = tx
                f_ref[s, 1, ci] = ty
                f_ref[s, 2, ci] = gw
                f_ref[s, 3, ci] = gh
                f_ref[s, 4, ci] = mk.astype(jnp.float32)


def _k_prep(tt):
    return pl.pallas_call(
        _prep_body,
        out_shape=(jax.ShapeDtypeStruct((3, 6, 15, 32, 128), jnp.int32),
                   jax.ShapeDtypeStruct((3, 5, 15, 32, 128), jnp.float32)),
        interpret=_INTERPRET,
    )(tt)


def _dense_body(x_ref, s_ref, d_ref, os_ref):
    rows = []
    for g in range(10):
        pr = 1.0 - x_ref[0, 5 + 8 * g]
        for k in range(1, 8):
            pr = pr * (1.0 - x_ref[0, 5 + 8 * g + k])
        rows.append(jnp.log(pr))
    acc = rows[0]
    for r in rows[1:]:
        acc = acc + r
    s_ref[0] = acc
    po = x_ref[0, 4]
    lo1 = jnp.log(1.0 - po)
    lo0 = jnp.log(po)
    d_ref[0] = lo1 - lo0
    os_ref[0, 0] = jnp.sum(lo1, axis=0)


def _k_dense(pred, s):
    hw = HWS[s]
    sub = hw // 128
    p = pred.reshape(B, 255, sub, 128)
    grid = (B * NA,)
    return pl.pallas_call(
        _dense_body,
        grid=grid,
        in_specs=[pl.BlockSpec((1, 85, sub, 128), lambda i: (i // 3, i % 3, 0, 0))],
        out_specs=(pl.BlockSpec((1, sub, 128), lambda i: (i, 0, 0)),
                   pl.BlockSpec((1, sub, 128), lambda i: (i, 0, 0)),
                   pl.BlockSpec((1, 1, 128), lambda i: (i, 0, 0))),
        out_shape=(jax.ShapeDtypeStruct((B * NA, sub, 128), jnp.float32),
                   jax.ShapeDtypeStruct((B * NA, sub, 128), jnp.float32),
                   jax.ShapeDtypeStruct((B * NA, 1, 128), jnp.float32)),
        interpret=_INTERPRET,
    )(p)


def _atan_pos(z):
    inv = z > 1.0
    zz = jnp.where(inv, 1.0 / z, z)
    x2 = zz * zz
    p = -0.0117212
    p = p * x2 + 0.0529126
    p = p * x2 - 0.1169414
    p = p * x2 + 0.1939339
    p = p * x2 - 0.3326221
    p = p * x2 + 0.9999791
    at = p * zz
    return jnp.where(inv, (math.pi / 2.0) - at, at)


def _entry_body(g_ref, f_ref, im_ref, val_ref, sidx_ref, sums_ref):
    for s in range(3):
        acc_sl1 = jnp.zeros((32, 128), jnp.float32)
        acc_bce = jnp.zeros((32, 128), jnp.float32)
        acc_cnt = jnp.zeros((32, 128), jnp.float32)
        for ci in range(15):
            a = ci % NA
            aw, ah = ANCH[s][a]
            p0 = g_ref[s, 0, ci]
            p1 = g_ref[s, 1, ci]
            p2 = g_ref[s, 2, ci]
            p3 = g_ref[s, 3, ci]
            pct = g_ref[s, 4, ci]
            sv = g_ref[s, 5, ci]
            tx = f_ref[s, 0, ci]
            ty = f_ref[s, 1, ci]
            tw = f_ref[s, 2, ci]
            th = f_ref[s, 3, ci]
            mk = f_ref[s, 4, ci]
            px = p0 * 2.0 - 0.5
            py = p1 * 2.0 - 0.5
            pw = (p2 * 2.0) ** 2 * aw
            ph = (p3 * 2.0) ** 2 * ah
            sl1 = jnp.zeros((32, 128), jnp.float32)
            for pv, tv in ((px, tx), (py, ty), (pw, tw), (ph, th)):
                d = jnp.abs(pv - tv)
                sl1 = sl1 + jnp.where(d < 1.0, 0.5 * d * d, d - 0.5)
            acc_sl1 = acc_sl1 + mk * sl1
            bce = -jnp.log(pct) + jnp.log(1.0 - pct) - sv
            acc_bce = acc_bce + mk * bce
            acc_cnt = acc_cnt + mk
            b1x1 = px - pw * 0.5
            b1x2 = px + pw * 0.5
            b1y1 = py - ph * 0.5
            b1y2 = py + ph * 0.5
            b2x1 = tx - tw * 0.5
            b2x2 = tx + tw * 0.5
            b2y1 = ty - th * 0.5
            b2y2 = ty + th * 0.5
            iw = jnp.maximum(jnp.minimum(b1x2, b2x2) - jnp.maximum(b1x1, b2x1), 0.0)
            ih = jnp.maximum(jnp.minimum(b1y2, b2y2) - jnp.maximum(b1y1, b2y1), 0.0)
            inter = iw * ih
            union = pw * ph + 1e-16 + tw * th - inter
            iou = inter / union
            cw = jnp.maximum(b1x2, b2x2) - jnp.minimum(b1x1, b2x1)
            ch = jnp.maximum(b1y2, b2y2) - jnp.minimum(b1y1, b2y1)
            c2 = cw * cw + ch * ch + 1e-16
            rho2 = ((b2x1 + b2x2 - b1x1 - b1x2) ** 2
                    + (b2y1 + b2y2 - b1y1 - b1y2) ** 2) * 0.25
            v = (4.0 / (math.pi ** 2)) * (_atan_pos(tw / th) - _atan_pos(pw / ph)) ** 2
            alpha = v / (1.0 - iou + v + 1e-16)
            ciou = iou - (rho2 / c2 + v * alpha)
            val_ref[s, ci] = 0.5 + 0.5 * jnp.maximum(ciou, 0.0)
            sidx_ref[s, ci] = jnp.where(mk > 0.5, im_ref[s, ci], DUMMY)
        sums_ref[s, 0] = jnp.sum(acc_sl1, axis=0)
        sums_ref[s, 1] = jnp.sum(acc_bce, axis=0)
        sums_ref[s, 2] = jnp.sum(acc_cnt, axis=0)
        for r in range(3, 8):
            sums_ref[s, r] = jnp.zeros((128,), jnp.float32)


def _k_entry(g, ft, im):
    return pl.pallas_call(
        _entry_body,
        out_shape=(jax.ShapeDtypeStruct((3, 15, 32, 128), jnp.float32),
                   jax.ShapeDtypeStruct((3, 15, 32, 128), jnp.int32),
                   jax.ShapeDtypeStruct((3, 8, 128), jnp.float32)),
        interpret=_INTERPRET,
    )(g, ft, im)


def _fin_body(tobj_ref, d0_ref, d1_ref, d2_ref, o0_ref, o1_ref, o2_ref,
              sums_ref, out_ref):
    r0 = 1536
    r1 = 1920
    st = [jnp.sum(tobj_ref[0:r0] * d0_ref[...]),
          jnp.sum(tobj_ref[r0:r1] * d1_ref[...]),
          jnp.sum(tobj_ref[r1:2016] * d2_ref[...])]
    osum = [jnp.sum(o0_ref[...]), jnp.sum(o1_ref[...]), jnp.sum(o2_ref[...])]
    lobj = jnp.float32(0.0)
    lbox = jnp.float32(0.0)
    lcls = jnp.float32(0.0)
    for s in range(3):
        lobj = lobj + BAL[s] * (-osum[s] + st[s]) / float(NPOS[s])
        sl1 = jnp.sum(sums_ref[s, 0])
        bce = jnp.sum(sums_ref[s, 1])
        cnt = jnp.sum(sums_ref[s, 2])
        den = jnp.maximum(cnt, 1.0)
        lbox = lbox + sl1 / (den * 4.0)
        lcls = lcls + bce / (den * float(NC))
    lbox = lbox * 0.05
    lobj = lobj * 1.4
    lcls = lcls * 0.5
    loss = (lbox + lobj + lcls) * float(B)
    out_ref[0] = jnp.full((128,), loss, jnp.float32)
    out_ref[1] = jnp.full((128,), lbox, jnp.float32)
    out_ref[2] = jnp.full((128,), lobj, jnp.float32)
    out_ref[3] = jnp.full((128,), lcls, jnp.float32)


def _k_fin(tobj2d, d0, d1, d2, o0, o1, o2, sums):
    return pl.pallas_call(
        _fin_body,
        out_shape=jax.ShapeDtypeStruct((4, 128), jnp.float32),
        interpret=_INTERPRET,
    )(tobj2d, d0, d1, d2, o0, o1, o2, sums)


def kernel(pred0, pred1, pred2, targets):
    preds = [pred0, pred1, pred2]
    tt = targets.T.reshape(6, 32, 128)
    idx, ft = _k_prep(tt)
    dense = [_k_dense(preds[s], s) for s in range(3)]
    sflat = jnp.concatenate([dense[s][0].reshape(-1) for s in range(3)])
    gs = []
    for s in range(3):
        pf = preds[s].reshape(-1)
        gb = pf[idx[s, 0:5].reshape(5, -1)]
        gm = sflat[idx[s, 5].reshape(1, -1)]
        gs.append(jnp.concatenate([gb, gm], axis=0))
    g = jnp.stack(gs).reshape(3, 6, 15, 32, 128)
    val, sidx, sums = _k_entry(g, ft, idx[:, 5])
    tobj = jnp.zeros((TOBJ_PAD,), jnp.float32)
    tobj = tobj.at[sidx.reshape(-1)].set(val.reshape(-1))
    tobj2d = tobj[:TOT].reshape(2016, 128)
    out = _k_fin(tobj2d,
                 dense[0][1].reshape(1536, 128),
                 dense[1][1].reshape(384, 128),
                 dense[2][1].reshape(96, 128),
                 dense[0][2].reshape(48, 128),
                 dense[1][2].reshape(48, 128),
                 dense[2][2].reshape(48, 128),
                 sums)
    return out[0, :1], out[1:4, 0]
